# baseline (speedup 1.0000x reference)
import functools

import jax
import jax.numpy as jnp
from jax import lax
from jax.experimental import pallas as pl
from jax.experimental.pallas import tpu as pltpu
from jax.experimental.pallas import tpu_sc as plsc

LORA_BATCH = 320
R_SMALL = 16
NUM_ADAPTERS = 80
D_OUT = 4096
N_OUT_ROWS = 128
K_DIM = NUM_ADAPTERS * R_SMALL
PAD_ROWS = 384

_sc_mesh = plsc.VectorSubcoreMesh(core_axis_name="c", subcore_axis_name="s")


@functools.partial(
    pl.kernel,
    mesh=_sc_mesh,
    out_type=jax.ShapeDtypeStruct((N_OUT_ROWS, K_DIM), jnp.float32),
    scratch_types=[
        pltpu.VMEM((16, 1, R_SMALL), jnp.float32),
        pltpu.VMEM((16,), jnp.int32),
        pltpu.VMEM((8, K_DIM), jnp.float32),
    ],
    compiler_params=pltpu.CompilerParams(needs_layout_passes=False),
)
def _sc_scatter(y_hbm, wids_hbm, s_hbm, yv, wv, acc):
    w = lax.axis_index("s") * 2 + lax.axis_index("c")
    is_large = w < 16
    is_small = jnp.logical_and(w >= 16, w < 24)
    ws = jnp.minimum(w - 16, 7)
    ibase = jnp.where(is_large, 16 * w, 256 + 8 * ws)
    nitems = jnp.where(is_large, 16, jnp.where(w < 24, 8, 0))
    nrows = jnp.where(is_large, 4, jnp.where(w < 24, 8, 0))

    @pl.when(is_large)
    def _():
        pltpu.sync_copy(y_hbm.at[pl.ds(ibase, 16)], yv)
        pltpu.sync_copy(wids_hbm.at[pl.ds(ibase, 16)], wv)

    @pl.when(is_small)
    def _():
        pltpu.sync_copy(y_hbm.at[pl.ds(ibase, 8)], yv.at[pl.ds(0, 8)])
        pltpu.sync_copy(wids_hbm.at[pl.ds(ibase, 8)], wv.at[pl.ds(0, 8)])

    zeros16 = jnp.zeros((R_SMALL,), jnp.float32)

    def _zero_body(i, carry):
        r = i // (K_DIM // (4 * R_SMALL))
        cchunk = i % (K_DIM // (4 * R_SMALL))
        for u in range(4):
            acc[r, pl.ds((4 * cchunk + u) * R_SMALL, R_SMALL)] = zeros16
        return carry

    lax.fori_loop(0, nrows * (K_DIM // (4 * R_SMALL)), _zero_body, 0)

    lane_iota = lax.iota(jnp.int32, 16)
    wvec = wv[...]

    def _item_body(j, carry):
        wid_b = wvec.at[jnp.full((16,), j, jnp.int32)].get(
            mode="promise_in_bounds")
        yj = yv[j, 0, :]
        r = jnp.where(is_large, j // 4, j)
        rvec = lax.broadcast(r, (16,))
        col = wid_b * R_SMALL + lane_iota
        plsc.addupdate_scatter(acc, [rvec, col], yj)
        return carry

    lax.fori_loop(0, nitems, _item_body, 0)

    @pl.when(is_large)
    def _():
        pltpu.sync_copy(acc.at[pl.ds(0, 4)], s_hbm.at[pl.ds(4 * w, 4)])

    @pl.when(is_small)
    def _():
        pltpu.sync_copy(acc.at[pl.ds(0, 8)], s_hbm.at[pl.ds(64 + 8 * ws, 8)])


def _mm_body(s_ref, b_ref, o_ref):
    acc = jnp.dot(
        s_ref[...],
        b_ref[...],
        preferred_element_type=jnp.float32,
    )
    o_ref[...] = acc * 2.0


_N_BLK = 4096


def _tc_matmul(s, b_flat):
    return pl.pallas_call(
        _mm_body,
        grid=(D_OUT // _N_BLK,),
        in_specs=[
            pl.BlockSpec((N_OUT_ROWS, K_DIM), lambda i: (0, 0)),
            pl.BlockSpec((K_DIM, _N_BLK), lambda i: (0, i)),
        ],
        out_specs=pl.BlockSpec((N_OUT_ROWS, _N_BLK), lambda i: (0, i)),
        out_shape=jax.ShapeDtypeStruct((N_OUT_ROWS, D_OUT), jnp.float32),
        compiler_params=pltpu.CompilerParams(
            allow_input_fusion=[True, True],
        ),
    )(s, b_flat)


@jax.jit
def kernel(y, wids, lora_B):
    y32 = y.astype(jnp.float32)
    s = _sc_scatter(y32, wids)
    out = _tc_matmul(
        s.astype(jnp.bfloat16),
        lora_B.reshape(K_DIM, D_OUT).astype(jnp.bfloat16),
    )
    return out.astype(jnp.float16).reshape(N_OUT_ROWS, 1, D_OUT)


# speedup vs baseline: 1.0200x; 1.0200x over previous
import functools

import jax
import jax.numpy as jnp
from jax import lax
from jax.experimental import pallas as pl
from jax.experimental.pallas import tpu as pltpu
from jax.experimental.pallas import tpu_sc as plsc

LORA_BATCH = 320
R_SMALL = 16
NUM_ADAPTERS = 80
D_OUT = 4096
N_OUT_ROWS = 128
K_DIM = NUM_ADAPTERS * R_SMALL
PAD_ROWS = 384

_sc_mesh = plsc.VectorSubcoreMesh(core_axis_name="c", subcore_axis_name="s")


@functools.partial(
    pl.kernel,
    mesh=_sc_mesh,
    out_type=jax.ShapeDtypeStruct((N_OUT_ROWS, K_DIM), jnp.float32),
    scratch_types=[
        pltpu.VMEM((16, 1, R_SMALL), jnp.float32),
        pltpu.VMEM((16,), jnp.int32),
        pltpu.VMEM((8, K_DIM), jnp.float32),
    ],
    compiler_params=pltpu.CompilerParams(
        needs_layout_passes=False,
        disable_bounds_checks=True,
        skip_device_barrier=True,
    ),
)
def _sc_scatter(y_hbm, wids_hbm, s_hbm, yv, wv, acc):
    w = lax.axis_index("s") * 2 + lax.axis_index("c")
    is_large = w < 16
    is_small = jnp.logical_and(w >= 16, w < 24)
    ws = jnp.minimum(w - 16, 7)
    ibase = jnp.where(is_large, 16 * w, 256 + 8 * ws)
    nitems = jnp.where(is_large, 16, jnp.where(w < 24, 8, 0))
    nrows = jnp.where(is_large, 4, jnp.where(w < 24, 8, 0))

    @pl.when(is_large)
    def _():
        pltpu.sync_copy(y_hbm.at[pl.ds(ibase, 16)], yv)
        pltpu.sync_copy(wids_hbm.at[pl.ds(ibase, 16)], wv)

    @pl.when(is_small)
    def _():
        pltpu.sync_copy(y_hbm.at[pl.ds(ibase, 8)], yv.at[pl.ds(0, 8)])
        pltpu.sync_copy(wids_hbm.at[pl.ds(ibase, 8)], wv.at[pl.ds(0, 8)])

    zeros16 = jnp.zeros((R_SMALL,), jnp.float32)

    def _zero_body(i, carry):
        r = i // (K_DIM // (4 * R_SMALL))
        cchunk = i % (K_DIM // (4 * R_SMALL))
        for u in range(4):
            acc[r, pl.ds((4 * cchunk + u) * R_SMALL, R_SMALL)] = zeros16
        return carry

    lax.fori_loop(0, nrows * (K_DIM // (4 * R_SMALL)), _zero_body, 0)

    lane_iota = lax.iota(jnp.int32, 16)
    wvec = wv[...]

    def _item_body(j, carry):
        wid_b = wvec.at[jnp.full((16,), j, jnp.int32)].get(
            mode="promise_in_bounds")
        yj = yv[j, 0, :]
        r = jnp.where(is_large, j // 4, j)
        rvec = lax.broadcast(r, (16,))
        col = wid_b * R_SMALL + lane_iota
        plsc.addupdate_scatter(acc, [rvec, col], yj)
        return carry

    lax.fori_loop(0, nitems, _item_body, 0)

    @pl.when(is_large)
    def _():
        pltpu.sync_copy(acc.at[pl.ds(0, 4)], s_hbm.at[pl.ds(4 * w, 4)])

    @pl.when(is_small)
    def _():
        pltpu.sync_copy(acc.at[pl.ds(0, 8)], s_hbm.at[pl.ds(64 + 8 * ws, 8)])


def _mm_body(s_ref, b_ref, o_ref):
    acc = jnp.dot(
        s_ref[...],
        b_ref[...],
        preferred_element_type=jnp.float32,
    )
    o_ref[...] = acc * 2.0


_N_BLK = 2048


def _tc_matmul(s, b_flat):
    return pl.pallas_call(
        _mm_body,
        grid=(D_OUT // _N_BLK,),
        in_specs=[
            pl.BlockSpec((N_OUT_ROWS, K_DIM), lambda i: (0, 0)),
            pl.BlockSpec((K_DIM, _N_BLK), lambda i: (0, i)),
        ],
        out_specs=pl.BlockSpec((N_OUT_ROWS, _N_BLK), lambda i: (0, i)),
        out_shape=jax.ShapeDtypeStruct((N_OUT_ROWS, D_OUT), jnp.float32),
        compiler_params=pltpu.CompilerParams(
            allow_input_fusion=[True, True],
        ),
    )(s, b_flat)


@jax.jit
def kernel(y, wids, lora_B):
    y32 = y.astype(jnp.float32)
    s = _sc_scatter(y32, wids)
    out = _tc_matmul(
        s.astype(jnp.bfloat16),
        lora_B.reshape(K_DIM, D_OUT).astype(jnp.bfloat16),
    )
    return out.astype(jnp.float16).reshape(N_OUT_ROWS, 1, D_OUT)


# speedup vs baseline: 1.0289x; 1.0087x over previous
import functools

import jax
import jax.numpy as jnp
from jax import lax
from jax.experimental import pallas as pl
from jax.experimental.pallas import tpu as pltpu
from jax.experimental.pallas import tpu_sc as plsc

LORA_BATCH = 320
R_SMALL = 16
NUM_ADAPTERS = 80
D_OUT = 4096
N_OUT_ROWS = 128
K_DIM = NUM_ADAPTERS * R_SMALL
PAD_ROWS = 384

_sc_mesh = plsc.VectorSubcoreMesh(core_axis_name="c", subcore_axis_name="s")


@functools.partial(
    pl.kernel,
    mesh=_sc_mesh,
    out_type=jax.ShapeDtypeStruct((N_OUT_ROWS, K_DIM), jnp.float32),
    scratch_types=[
        pltpu.VMEM((16, 1, R_SMALL), jnp.float32),
        pltpu.VMEM((16,), jnp.int32),
        pltpu.VMEM((8, K_DIM), jnp.float32),
    ],
    compiler_params=pltpu.CompilerParams(
        needs_layout_passes=False,
        disable_bounds_checks=True,
        skip_device_barrier=True,
    ),
)
def _sc_scatter(y_hbm, wids_hbm, s_hbm, yv, wv, acc):
    w = lax.axis_index("s") * 2 + lax.axis_index("c")
    is_large = w < 16
    is_small = jnp.logical_and(w >= 16, w < 24)
    ws = jnp.minimum(w - 16, 7)
    ibase = jnp.where(is_large, 16 * w, 256 + 8 * ws)
    nitems = jnp.where(is_large, 16, jnp.where(w < 24, 8, 0))
    nrows = jnp.where(is_large, 4, jnp.where(w < 24, 8, 0))

    @pl.when(is_large)
    def _():
        pltpu.sync_copy(y_hbm.at[pl.ds(ibase, 16)], yv)
        pltpu.sync_copy(wids_hbm.at[pl.ds(ibase, 16)], wv)

    @pl.when(is_small)
    def _():
        pltpu.sync_copy(y_hbm.at[pl.ds(ibase, 8)], yv.at[pl.ds(0, 8)])
        pltpu.sync_copy(wids_hbm.at[pl.ds(ibase, 8)], wv.at[pl.ds(0, 8)])

    zeros16 = jnp.zeros((R_SMALL,), jnp.float32)

    def _zero_body(i, carry):
        r = i // (K_DIM // (4 * R_SMALL))
        cchunk = i % (K_DIM // (4 * R_SMALL))
        for u in range(4):
            acc[r, pl.ds((4 * cchunk + u) * R_SMALL, R_SMALL)] = zeros16
        return carry

    lax.fori_loop(0, nrows * (K_DIM // (4 * R_SMALL)), _zero_body, 0)

    lane_iota = lax.iota(jnp.int32, 16)
    wvec = wv[...]

    def _item_body(j, carry):
        wid_b = wvec.at[jnp.full((16,), j, jnp.int32)].get(
            mode="promise_in_bounds")
        yj = yv[j, 0, :]
        r = jnp.where(is_large, j // 4, j)
        rvec = lax.broadcast(r, (16,))
        col = wid_b * R_SMALL + lane_iota
        plsc.addupdate_scatter(acc, [rvec, col], yj)
        return carry

    lax.fori_loop(0, nitems, _item_body, 0)

    @pl.when(is_large)
    def _():
        pltpu.sync_copy(acc.at[pl.ds(0, 4)], s_hbm.at[pl.ds(4 * w, 4)])

    @pl.when(is_small)
    def _():
        pltpu.sync_copy(acc.at[pl.ds(0, 8)], s_hbm.at[pl.ds(64 + 8 * ws, 8)])


def _mm_body(s_ref, b_ref, o_ref):
    acc = jnp.dot(
        s_ref[...],
        b_ref[...],
        preferred_element_type=jnp.float32,
    )
    o_ref[...] = acc * 2.0


_N_BLK = 2048


def _tc_matmul(s, b_flat):
    return pl.pallas_call(
        _mm_body,
        grid=(D_OUT // _N_BLK,),
        in_specs=[
            pl.BlockSpec((N_OUT_ROWS, K_DIM), lambda i: (0, 0)),
            pl.BlockSpec((K_DIM, _N_BLK), lambda i: (0, i)),
        ],
        out_specs=pl.BlockSpec((N_OUT_ROWS, _N_BLK), lambda i: (0, i)),
        out_shape=jax.ShapeDtypeStruct((N_OUT_ROWS, D_OUT), jnp.float32),
        compiler_params=pltpu.CompilerParams(
            allow_input_fusion=[True, True],
        ),
    )(s, b_flat)


@jax.jit
def kernel(y, wids, lora_B):
    b_bf = lora_B.reshape(K_DIM, D_OUT).astype(jnp.bfloat16)
    y32 = y.astype(jnp.float32)
    s = _sc_scatter(y32, wids)
    out = _tc_matmul(s.astype(jnp.bfloat16), b_bf)
    return out.astype(jnp.float16).reshape(N_OUT_ROWS, 1, D_OUT)
